# Initial kernel scaffold; baseline (speedup 1.0000x reference)
import jax
import jax.numpy as jnp
from jax.experimental import pallas as pl


def kernel(H, col, row, W_theta, b_theta, W_out, b_out, gamma, beta):
    raise NotImplementedError("write your pallas kernel here")


# speedup vs baseline: 6.3907x; 6.3907x over previous
import functools

import jax
import jax.numpy as jnp
from jax import lax
from jax.experimental import pallas as pl
from jax.experimental.pallas import tpu as pltpu
from jax.experimental.pallas import tpu_sc as plsc

N = 10000
D = 128
K = 33

GN = 8
GE = GN * K
NG = N // GN
NC = 2
NS = 16
NW = NC * NS
TPW = (NG + NW - 1) // NW
ECH = 88


def _prep_body(h_ref, wt_ref, bt_ref, g_ref, b_ref, hn_ref, p_ref, q_ref):
    h = h_ref[...]
    mu = jnp.mean(h, axis=0, keepdims=True)
    var = jnp.mean((h - mu) ** 2, axis=0, keepdims=True)
    rstd = lax.rsqrt(var + 1e-5)
    hn = (h - mu) * (rstd * g_ref[...]) + b_ref[...]
    hn_ref[...] = hn
    wt = wt_ref[...]
    m = lax.dot_general(wt, wt, (((1,), (1,)), ((), ())),
                        preferred_element_type=jnp.float32)
    p_ref[...] = jnp.dot(hn, m, preferred_element_type=jnp.float32)
    bt = bt_ref[...]
    wb = lax.dot_general(wt, bt, (((1,), (1,)), ((), ())),
                         preferred_element_type=jnp.float32)
    q = jnp.dot(hn, wb, preferred_element_type=jnp.float32)
    q_ref[...] = q + 0.5 * jnp.sum(bt * bt)


def _out_body(y_ref, wo_ref, bo_ref, o_ref):
    z = jnp.dot(y_ref[...], wo_ref[...],
                preferred_element_type=jnp.float32) + bo_ref[...]
    o_ref[...] = jnp.where(z >= 0, z, 0.01 * z)


def _sc_body(hn_hbm, p_hbm, q_hbm, col_hbm, a_hbm, y_hbm,
             idx_v, rows_v, qc_v, ps_v, qs_v, e_v, a_v, y_v, sem):
    wid = lax.axis_index("s") * NC + lax.axis_index("c")
    kio = lax.iota(jnp.int32, 16)

    def trip(t, _):
        g = t * NW + wid

        @pl.when(g < NG)
        def _():
            be = g * GE
            bn = g * GN
            pltpu.sync_copy(col_hbm.at[pl.ds(be, GE)], idx_v)
            cps = []
            for c in range(GE // ECH):
                sl = pl.ds(c * ECH, ECH)
                cps.append(pltpu.async_copy(
                    hn_hbm.at[idx_v.at[sl]], rows_v.at[sl], sem))
                cps.append(pltpu.async_copy(
                    q_hbm.at[idx_v.at[sl]], qc_v.at[sl], sem))
            pltpu.sync_copy(p_hbm.at[pl.ds(bn, GN)], ps_v)
            pltpu.sync_copy(q_hbm.at[pl.ds(bn, GN)], qs_v.at[pl.ds(0, GN)])
            for cp in cps:
                cp.wait()

            for n in range(GN):
                row0 = n * K
                p = [ps_v[n, pl.ds(16 * j, 16)] for j in range(D // 16)]

                qi = plsc.load_gather(qs_v, [jnp.full((16,), n, jnp.int32)])
                sv = jnp.zeros((16,), jnp.float32)
                evs = []
                for b in range(3):
                    klen = 16 if b < 2 else K - 32

                    def dot_k(kk, dv, _b=b):
                        e = row0 + _b * 16 + kk
                        part = rows_v[e, pl.ds(0, 16)] * p[0]
                        for j in range(1, D // 16):
                            part += rows_v[e, pl.ds(16 * j, 16)] * p[j]
                        d = jnp.full((16,), jnp.sum(part), jnp.float32)
                        return jnp.where(kio == kk, d, dv)

                    dv = lax.fori_loop(0, klen, dot_k,
                                       jnp.zeros((16,), jnp.float32),
                                       unroll=4)
                    ki = b * 16 + kio
                    valid = ki < K
                    qcg = plsc.load_gather(
                        qc_v, [jnp.minimum(row0 + ki, GE - 1)])
                    ev = jnp.exp(1.0 / (1.0 + jnp.exp(-(dv + qcg + qi))))
                    ev = jnp.where(valid, ev, 0.0)
                    e_v[pl.ds(16 * b, 16)] = ev
                    evs.append(ev)
                    sv = sv + ev
                rsv = 1.0 / jnp.full((16,), jnp.sum(sv), jnp.float32)
                for b in range(3):
                    ki = b * 16 + kio
                    plsc.store_scatter(a_v, [jnp.minimum(row0 + ki, GE - 1)],
                                       evs[b] * rsv, mask=ki < K)

                def agg_k(k, acc):
                    e = row0 + k
                    eb = plsc.load_gather(e_v, [jnp.full((16,), k, jnp.int32)])
                    return tuple(
                        acc[j] + eb * rows_v[e, pl.ds(16 * j, 16)]
                        for j in range(D // 16))

                acc = lax.fori_loop(
                    0, K, agg_k,
                    tuple(jnp.zeros((16,), jnp.float32)
                          for _ in range(D // 16)),
                    unroll=4)
                for j in range(D // 16):
                    y_v[n, pl.ds(16 * j, 16)] = acc[j] * rsv

            pltpu.sync_copy(a_v, a_hbm.at[pl.ds(be, GE)])
            pltpu.sync_copy(y_v, y_hbm.at[pl.ds(bn, GN)])

        return 0

    lax.fori_loop(0, TPW, trip, 0)


@jax.jit
def kernel(H, col, row, W_theta, b_theta, W_out, b_out, gamma, beta):
    del row
    hn, p, q = pl.pallas_call(
        _prep_body,
        out_shape=[
            jax.ShapeDtypeStruct((N, D), jnp.float32),
            jax.ShapeDtypeStruct((N, D), jnp.float32),
            jax.ShapeDtypeStruct((N, 1), jnp.float32),
        ],
    )(H, W_theta, b_theta.reshape(1, D), gamma.reshape(1, D),
      beta.reshape(1, D))

    mesh = plsc.VectorSubcoreMesh(core_axis_name="c", subcore_axis_name="s",
                                  num_cores=NC, num_subcores=NS)
    sc = pl.kernel(
        _sc_body,
        out_type=[
            jax.ShapeDtypeStruct((N * K,), jnp.float32),
            jax.ShapeDtypeStruct((N, D), jnp.float32),
        ],
        mesh=mesh,
        compiler_params=pltpu.CompilerParams(needs_layout_passes=False),
        scratch_types=[
            pltpu.VMEM((GE,), jnp.int32),
            pltpu.VMEM((GE, D), jnp.float32),
            pltpu.VMEM((GE,), jnp.float32),
            pltpu.VMEM((GN, D), jnp.float32),
            pltpu.VMEM((16,), jnp.float32),
            pltpu.VMEM((48,), jnp.float32),
            pltpu.VMEM((GE,), jnp.float32),
            pltpu.VMEM((GN, D), jnp.float32),
            pltpu.SemaphoreType.DMA,
        ],
    )
    a_flat, y = sc(hn, p, q.reshape(N), col)

    out2 = pl.pallas_call(
        _out_body,
        out_shape=jax.ShapeDtypeStruct((N, D), jnp.float32),
    )(y, W_out, b_out.reshape(1, D))

    return out2, a_flat.reshape(N, K, 1)
